# Initial kernel scaffold; baseline (speedup 1.0000x reference)
import math

import functools
import numpy as np
import jax
import jax.numpy as jnp
from jax import lax
from jax.experimental import pallas as pl
from jax.experimental.pallas import tpu as pltpu
from jax.experimental.pallas import tpu_sc as plsc

VOCAB = 100000
EMB_DIM = 64
BATCH = 4096
SEQ = 200
SCALE = math.sqrt(EMB_DIM)

NUM_CORES = 2
NUM_SUBCORES = 16
NUM_WORKERS = NUM_CORES * NUM_SUBCORES
ROWS_PER_WORKER = BATCH // NUM_WORKERS

IDX_SPLIT = 2
IDX_CHUNK = SEQ // IDX_SPLIT


def _sinusoidal_pe(seq_len, d_model):
    pos = np.arange(seq_len, dtype=np.float32)[:, None]
    div = np.exp(
        np.arange(0, d_model, 2, dtype=np.float32) * (-math.log(10000.0) / d_model)
    )
    pe = np.zeros((seq_len, d_model), dtype=np.float32)
    pe[:, 0::2] = np.sin(pos * div)
    pe[:, 1::2] = np.cos(pos * div)
    return pe


_PE = jnp.asarray(_sinusoidal_pe(SEQ, EMB_DIM))


@functools.partial(
    pl.kernel,
    out_type=jax.ShapeDtypeStruct((BATCH, SEQ, EMB_DIM), jnp.float32),
    mesh=plsc.VectorSubcoreMesh(core_axis_name="c", subcore_axis_name="s"),
    scratch_types=[
        pltpu.VMEM((IDX_SPLIT, IDX_CHUNK), jnp.int32),
        pltpu.VMEM((SEQ, EMB_DIM), jnp.float32),
        pltpu.VMEM((SEQ, EMB_DIM), jnp.float32),
        pltpu.SemaphoreType.DMA,
    ],
)
def _emb_kernel(idx_hbm, table_hbm, pe_hbm, out_hbm, idx_v, rows_v, pe_v, sem):
    wid = lax.axis_index("s") * NUM_CORES + lax.axis_index("c")
    base = wid * ROWS_PER_WORKER

    pltpu.sync_copy(pe_hbm, pe_v)

    def row_body(r, carry):
        row = base + r
        pltpu.sync_copy(idx_hbm.at[row], idx_v)
        copies = []
        for j in range(IDX_SPLIT):
            copies.append(
                pltpu.async_copy(
                    table_hbm.at[idx_v.at[j]],
                    rows_v.at[pl.ds(j * IDX_CHUNK, IDX_CHUNK)],
                    sem,
                )
            )
        for cp in copies:
            cp.wait()

        def pos_body(p, c2):
            for c in range(EMB_DIM // 16):
                sl = pl.ds(c * 16, 16)
                rows_v[p, sl] = rows_v[p, sl] * SCALE + pe_v[p, sl]
            return c2

        lax.fori_loop(0, SEQ, pos_body, 0, unroll=False)
        pltpu.sync_copy(rows_v, out_hbm.at[row])
        return carry

    lax.fori_loop(0, ROWS_PER_WORKER, row_body, 0, unroll=False)


def kernel(output, emb_table):
    idx = output.astype(jnp.int32).reshape(BATCH, IDX_SPLIT, IDX_CHUNK)
    return _emb_kernel(idx, emb_table, _PE)


# speedup vs baseline: 2.9915x; 2.9915x over previous
import math

import functools
import numpy as np
import jax
import jax.numpy as jnp
from jax import lax
from jax.experimental import pallas as pl
from jax.experimental.pallas import tpu as pltpu
from jax.experimental.pallas import tpu_sc as plsc

VOCAB = 100000
EMB_DIM = 64
BATCH = 4096
SEQ = 200
SCALE = math.sqrt(EMB_DIM)

NUM_CORES = 2
NUM_SUBCORES = 16
NUM_WORKERS = NUM_CORES * NUM_SUBCORES
ROWS_PER_WORKER = BATCH // NUM_WORKERS

IDX_SPLIT = 2
IDX_CHUNK = SEQ // IDX_SPLIT


def _sinusoidal_pe(seq_len, d_model):
    pos = np.arange(seq_len, dtype=np.float32)[:, None]
    div = np.exp(
        np.arange(0, d_model, 2, dtype=np.float32) * (-math.log(10000.0) / d_model)
    )
    pe = np.zeros((seq_len, d_model), dtype=np.float32)
    pe[:, 0::2] = np.sin(pos * div)
    pe[:, 1::2] = np.cos(pos * div)
    return pe


_PE_NP = _sinusoidal_pe(SEQ, EMB_DIM)


@functools.partial(
    pl.kernel,
    out_type=jax.ShapeDtypeStruct((BATCH, SEQ, EMB_DIM), jnp.float32),
    mesh=plsc.VectorSubcoreMesh(core_axis_name="c", subcore_axis_name="s"),
    compiler_params=pltpu.CompilerParams(use_tc_tiling_on_sc=False),
    scratch_types=[
        pltpu.VMEM((IDX_SPLIT, IDX_CHUNK), jnp.int32),
        pltpu.VMEM((SEQ, EMB_DIM), jnp.float32),
        pltpu.VMEM((SEQ, EMB_DIM), jnp.float32),
        pltpu.SemaphoreType.DMA,
    ],
)
def _emb_kernel(idx_hbm, table_hbm, pe_hbm, out_hbm, idx_v, rows_v, pe_v, sem):
    wid = lax.axis_index("s") * NUM_CORES + lax.axis_index("c")
    base = wid * ROWS_PER_WORKER

    pltpu.sync_copy(pe_hbm, pe_v)

    def row_body(r, carry):
        row = base + r
        pltpu.sync_copy(idx_hbm.at[row], idx_v)
        copies = []
        for j in range(IDX_SPLIT):
            copies.append(
                pltpu.async_copy(
                    table_hbm.at[idx_v.at[j]],
                    rows_v.at[pl.ds(j * IDX_CHUNK, IDX_CHUNK)],
                    sem,
                )
            )
        for cp in copies:
            cp.wait()

        def pos_body(p, c2):
            for c in range(EMB_DIM // 16):
                sl = pl.ds(c * 16, 16)
                rows_v[p, sl] = rows_v[p, sl] * SCALE + pe_v[p, sl]
            return c2

        lax.fori_loop(0, SEQ, pos_body, 0, unroll=False)
        pltpu.sync_copy(rows_v, out_hbm.at[row])
        return carry

    lax.fori_loop(0, ROWS_PER_WORKER, row_body, 0, unroll=False)


def kernel(output, emb_table):
    idx = output.astype(jnp.int32).reshape(BATCH, IDX_SPLIT, IDX_CHUNK)
    return _emb_kernel(idx, emb_table, jnp.asarray(_PE_NP))


# speedup vs baseline: 3.8673x; 1.2928x over previous
import math

import functools
import numpy as np
import jax
import jax.numpy as jnp
from jax import lax
from jax.experimental import pallas as pl
from jax.experimental.pallas import tpu as pltpu
from jax.experimental.pallas import tpu_sc as plsc

VOCAB = 100000
EMB_DIM = 64
BATCH = 4096
SEQ = 200
SCALE = math.sqrt(EMB_DIM)

NUM_CORES = 2
NUM_SUBCORES = 16
NUM_WORKERS = NUM_CORES * NUM_SUBCORES
ROWS_PER_WORKER = BATCH // NUM_WORKERS

NBUF = 4
ITERS = ROWS_PER_WORKER // NBUF

IDX_SPLIT = 2
IDX_CHUNK = SEQ // IDX_SPLIT


def _sinusoidal_pe(seq_len, d_model):
    pos = np.arange(seq_len, dtype=np.float32)[:, None]
    div = np.exp(
        np.arange(0, d_model, 2, dtype=np.float32) * (-math.log(10000.0) / d_model)
    )
    pe = np.zeros((seq_len, d_model), dtype=np.float32)
    pe[:, 0::2] = np.sin(pos * div)
    pe[:, 1::2] = np.cos(pos * div)
    return pe


_PE_NP = _sinusoidal_pe(SEQ, EMB_DIM)


@functools.partial(
    pl.kernel,
    out_type=jax.ShapeDtypeStruct((BATCH, SEQ, EMB_DIM), jnp.float32),
    mesh=plsc.VectorSubcoreMesh(core_axis_name="c", subcore_axis_name="s"),
    compiler_params=pltpu.CompilerParams(use_tc_tiling_on_sc=False),
    scratch_types=(
        [pltpu.VMEM((IDX_SPLIT, IDX_CHUNK), jnp.int32) for _ in range(NBUF)]
        + [pltpu.VMEM((SEQ, EMB_DIM), jnp.float32) for _ in range(NBUF)]
        + [pltpu.VMEM((SEQ, EMB_DIM), jnp.float32)]
        + [pltpu.SemaphoreType.DMA for _ in range(2 * NBUF)]
    ),
)
def _emb_kernel(idx_hbm, table_hbm, pe_hbm, out_hbm, *refs):
    idx_v = refs[0:NBUF]
    rows_v = refs[NBUF : 2 * NBUF]
    pe_v = refs[2 * NBUF]
    gsem = refs[2 * NBUF + 1 : 2 * NBUF + 1 + NBUF]
    wsem = refs[2 * NBUF + 1 + NBUF :]

    wid = lax.axis_index("s") * NUM_CORES + lax.axis_index("c")
    base = wid * ROWS_PER_WORKER

    def issue_gather(b, row):
        pltpu.sync_copy(idx_hbm.at[row], idx_v[b])
        for j in range(IDX_SPLIT):
            pltpu.async_copy(
                table_hbm.at[idx_v[b].at[j]],
                rows_v[b].at[pl.ds(j * IDX_CHUNK, IDX_CHUNK)],
                gsem[b],
            )

    def wait_gather(b):
        for j in range(IDX_SPLIT):
            pltpu.make_async_copy(
                table_hbm.at[idx_v[b].at[j]],
                rows_v[b].at[pl.ds(j * IDX_CHUNK, IDX_CHUNK)],
                gsem[b],
            ).wait()

    def issue_wb(b, row):
        pltpu.async_copy(rows_v[b], out_hbm.at[row], wsem[b])

    def wait_wb(b, row):
        pltpu.make_async_copy(rows_v[b], out_hbm.at[row], wsem[b]).wait()

    def refill(b, old_row, new_row):
        wait_wb(b, old_row)
        issue_gather(b, new_row)

    def compute(b):
        rows = rows_v[b]

        def body(i, c):
            for k in range(4):
                p = 4 * i + k
                for c4 in range(EMB_DIM // 16):
                    sl = pl.ds(c4 * 16, 16)
                    rows[p, sl] = rows[p, sl] * SCALE + pe_v[p, sl]
            return c

        lax.fori_loop(0, SEQ // 4, body, 0, unroll=False)

    pltpu.sync_copy(pe_hbm, pe_v)
    for b in range(NBUF):
        issue_gather(b, base + b)

    def outer(g, carry):
        r = base + NBUF * g

        @pl.when(g >= 1)
        def _():
            refill(NBUF - 1, r - 1, r + NBUF - 1)

        for b in range(NBUF):
            wait_gather(b)
            compute(b)
            issue_wb(b, r + b)
            if b >= 1:

                @pl.when(g <= ITERS - 2)
                def _():
                    refill(b - 1, r + b - 1, r + b - 1 + NBUF)

        return carry

    lax.fori_loop(0, ITERS, outer, 0, unroll=False)

    last = base + ROWS_PER_WORKER - NBUF
    for b in range(NBUF):
        wait_wb(b, last + b)


def kernel(output, emb_table):
    idx = output.astype(jnp.int32).reshape(BATCH, IDX_SPLIT, IDX_CHUNK)
    return _emb_kernel(idx, emb_table, jnp.asarray(_PE_NP))
